# baseline (speedup 1.0000x reference)
import functools

import jax
import jax.numpy as jnp
from jax import lax
from jax.experimental import pallas as pl
from jax.experimental.pallas import tpu as pltpu
from jax.experimental.pallas import tpu_sc as plsc

_B, _C, _H, _W = 32, 384, 64, 64
_R = _B * _H * _W
_NC, _NS = 2, 16
_NW = _NC * _NS
_RPW = _R // _NW
_RCH = 128
_NCHUNK = _RPW // _RCH
_NBUF = 2
_LANES = 16
_NK = _C // _LANES

_mesh = plsc.VectorSubcoreMesh(core_axis_name="c", subcore_axis_name="s")


def _reverse_lanes(buf):
    def body(r, _):
        for k in range(_NK // 2):
            k2 = _NK - 1 - k
            sa = pl.ds(k * _LANES, _LANES)
            sb = pl.ds(k2 * _LANES, _LANES)
            va = buf[r, sa]
            vb = buf[r, sb]
            buf[r, sa] = lax.rev(vb, (0,))
            buf[r, sb] = lax.rev(va, (0,))
        return 0

    lax.fori_loop(0, _RCH, body, 0)


@functools.partial(
    pl.kernel,
    out_type=jax.ShapeDtypeStruct((_R, _C), jnp.float32),
    mesh=_mesh,
    scratch_types=[
        pltpu.VMEM((_RCH, _C), jnp.float32),
        pltpu.VMEM((_RCH, _C), jnp.float32),
        pltpu.SemaphoreType.DMA,
        pltpu.SemaphoreType.DMA,
        pltpu.SemaphoreType.DMA,
        pltpu.SemaphoreType.DMA,
    ],
)
def _reverse_rows(x_hbm, out_hbm, buf0, buf1,
                  rs0, rs1, ws0, ws1):
    wid = lax.axis_index("s") * _NC + lax.axis_index("c")
    base = wid * _RPW
    bufs = (buf0, buf1)
    rsems = (rs0, rs1)
    wsems = (ws0, ws1)

    def rows(j):
        return pl.ds(base + j * _RCH, _RCH)

    reads = [None] * _NCHUNK
    writes = [None] * _NCHUNK
    for j in range(min(1, _NCHUNK)):
        reads[j] = pltpu.async_copy(x_hbm.at[rows(j)], bufs[j % _NBUF],
                                    rsems[j % _NBUF])
    for j in range(_NCHUNK):
        b = j % _NBUF
        reads[j].wait()
        _reverse_lanes(bufs[b])
        writes[j] = pltpu.async_copy(bufs[b], out_hbm.at[rows(j)],
                                     wsems[b])
        nxt = j + 1
        if nxt < _NCHUNK:
            nb = nxt % _NBUF
            if nxt >= _NBUF:
                writes[nxt - _NBUF].wait()
            reads[nxt] = pltpu.async_copy(x_hbm.at[rows(nxt)], bufs[nb],
                                          rsems[nb])
    for j in range(max(0, _NCHUNK - _NBUF), _NCHUNK):
        writes[j].wait()


def kernel(x):
    xt = jnp.transpose(x, (0, 2, 3, 1)).reshape(_R, _C)
    out = _reverse_rows(xt)
    return jnp.transpose(out.reshape(_B, _H, _W, _C), (0, 3, 1, 2))


# speedup vs baseline: 1.1427x; 1.1427x over previous
import functools

import jax
import jax.numpy as jnp
from jax import lax
from jax.experimental import pallas as pl
from jax.experimental.pallas import tpu as pltpu
from jax.experimental.pallas import tpu_sc as plsc

_B, _C, _H, _W = 32, 384, 64, 64
_R = _B * _H * _W
_NC, _NS = 2, 16
_NW = _NC * _NS
_RPW = _R // _NW
_RCH = 64
_NCHUNK = _RPW // _RCH
_NBUF = 3
_LANES = 16
_NK = _C // _LANES

_mesh = plsc.VectorSubcoreMesh(core_axis_name="c", subcore_axis_name="s")


def _reverse_lanes(buf):
    def body(r, _):
        for k in range(_NK // 2):
            k2 = _NK - 1 - k
            sa = pl.ds(k * _LANES, _LANES)
            sb = pl.ds(k2 * _LANES, _LANES)
            va = buf[r, sa]
            vb = buf[r, sb]
            buf[r, sa] = lax.rev(vb, (0,))
            buf[r, sb] = lax.rev(va, (0,))
        return 0

    lax.fori_loop(0, _RCH, body, 0)


@functools.partial(
    pl.kernel,
    out_type=jax.ShapeDtypeStruct((_R, _C), jnp.float32),
    mesh=_mesh,
    scratch_types=[
        pltpu.VMEM((_RCH, _C), jnp.float32),
        pltpu.VMEM((_RCH, _C), jnp.float32),
        pltpu.VMEM((_RCH, _C), jnp.float32),
        pltpu.SemaphoreType.DMA,
        pltpu.SemaphoreType.DMA,
        pltpu.SemaphoreType.DMA,
        pltpu.SemaphoreType.DMA,
        pltpu.SemaphoreType.DMA,
        pltpu.SemaphoreType.DMA,
    ],
)
def _reverse_rows(x_hbm, out_hbm, buf0, buf1, buf2,
                  rs0, rs1, rs2, ws0, ws1, ws2):
    wid = lax.axis_index("s") * _NC + lax.axis_index("c")
    base = wid * _RPW
    bufs = (buf0, buf1, buf2)
    rsems = (rs0, rs1, rs2)
    wsems = (ws0, ws1, ws2)

    def rows(j):
        return pl.ds(base + j * _RCH, _RCH)

    reads = [None] * _NCHUNK
    writes = [None] * _NCHUNK
    for j in range(min(2, _NCHUNK)):
        reads[j] = pltpu.async_copy(x_hbm.at[rows(j)], bufs[j % _NBUF],
                                    rsems[j % _NBUF])
    for j in range(_NCHUNK):
        b = j % _NBUF
        reads[j].wait()
        _reverse_lanes(bufs[b])
        writes[j] = pltpu.async_copy(bufs[b], out_hbm.at[rows(j)],
                                     wsems[b])
        nxt = j + 2
        if nxt < _NCHUNK:
            nb = nxt % _NBUF
            if nxt >= _NBUF:
                writes[nxt - _NBUF].wait()
            reads[nxt] = pltpu.async_copy(x_hbm.at[rows(nxt)], bufs[nb],
                                          rsems[nb])
    for j in range(_NCHUNK - _NBUF, _NCHUNK):
        if j >= 0:
            writes[j].wait()


def kernel(x):
    xt = jnp.transpose(x, (0, 2, 3, 1)).reshape(_R, _C)
    out = _reverse_rows(xt)
    return jnp.transpose(out.reshape(_B, _H, _W, _C), (0, 3, 1, 2))
